# baseline (speedup 1.0000x reference)
import jax
import jax.numpy as jnp
import numpy as np
from jax import lax
from jax.experimental import pallas as pl
from jax.experimental.pallas import tpu as pltpu
from jax.experimental.pallas import tpu_sc as plsc

N = 10000
E = 320000
D = 128
DK = 64
NB = 16
NL = 3
RMAX = 5.0
NSH = 9

NW = 32
BLK = 80
BB = 12800
NBB = BB // BLK
NSEG = 312
ASZ = 328

_f32 = jnp.float32
_i32 = jnp.int32

_mesh = plsc.VectorSubcoreMesh(core_axis_name="c", subcore_axis_name="s")
_scparams = pltpu.CompilerParams(needs_layout_passes=False)
_DEF = lax.Precision.DEFAULT

_C0 = 0.28209479177387814
_C1 = 0.4886025119029199
_C2A = 1.0925484305920792
_C2B = 0.31539156525252005
_C2C = 0.5462742152960396

_F1 = (0, 1, 2, 3, 1, 2, 3, 1, 1, 2, -1, -1, -1, -1, -1, -1)
_F2 = (0, 0, 0, 0, 1, 2, 3, 2, 3, 3, -1, -1, -1, -1, -1, -1)

_CM = np.zeros((16, 9), np.float32)
_CM[0, 0] = _C0
_CM[2, 1] = _C1
_CM[3, 2] = _C1
_CM[1, 3] = _C1
_CM[7, 4] = _C2A
_CM[9, 5] = _C2A
_CM[6, 6] = 3.0 * _C2B
_CM[0, 6] = -_C2B
_CM[8, 7] = _C2A
_CM[4, 8] = _C2C
_CM[5, 8] = -_C2C


def _wid():
    return lax.axis_index("s") * 2 + lax.axis_index("c")


def _rsqrt(d2):
    y = plsc.bitcast(jnp.int32(0x5F3759DF) - (plsc.bitcast(d2, _i32) >> 1),
                     _f32)
    for _ in range(3):
        y = y * (1.5 - 0.5 * d2 * y * y)
    return y


def _mask16(iot, sel, code):
    lanes = [k for k, s in enumerate(sel) if s == code]
    m = iot == lanes[0]
    for k in lanes[1:]:
        m = m | (iot == k)
    return m


def _bucket_body(src, dst, beid, bsrc, bdst, counts, ebuf, sbufL, dbufL,
                 sblk, dblk, cbuf):
    wid = _wid()
    lo = wid * NSEG
    hi = lo + NSEG + jnp.where(wid == NW - 1, 16, 0)
    iot = lax.iota(_i32, 16)
    lane0 = iot == 0
    zi = jnp.zeros((16,), _i32)

    def zero(i, carry):
        ebuf[pl.ds(i * 16, 16)] = zi
        sbufL[pl.ds(i * 16, 16)] = zi
        dbufL[pl.ds(i * 16, 16)] = zi
        return carry

    lax.fori_loop(0, (BB + 16) // 16, zero, 0)

    def blk(i, cnt):
        e0 = i * BLK
        pltpu.sync_copy(src.at[pl.ds(e0, BLK)], sblk)
        pltpu.sync_copy(dst.at[pl.ds(e0, BLK)], dblk)
        for j in range(BLK // 16):
            sv = sblk[pl.ds(j * 16, 16)]
            dv = dblk[pl.ds(j * 16, 16)]
            mask = ((dv >= lo) & (dv < hi)).astype(_i32)
            for e2 in range(16):
                m = mask[e2] == 1
                eid = e0 + j * 16 + e2
                cv = jnp.full((16,), cnt, _i32)

                @pl.when(m)
                def _store():
                    plsc.store_scatter(ebuf, [cv],
                                       jnp.full((16,), eid, _i32),
                                       mask=lane0)
                    plsc.store_scatter(sbufL, [cv],
                                       jnp.full((16,), sv[e2], _i32),
                                       mask=lane0)
                    plsc.store_scatter(dbufL, [cv],
                                       jnp.full((16,), dv[e2], _i32),
                                       mask=lane0)

                cnt = jnp.where(m, cnt + 1, cnt)
        return cnt

    cnt = lax.fori_loop(0, E // BLK, blk, jnp.int32(0))
    pltpu.sync_copy(ebuf.at[pl.ds(0, BB)], beid.at[pl.ds(wid * BB, BB)])
    pltpu.sync_copy(sbufL.at[pl.ds(0, BB)], bsrc.at[pl.ds(wid * BB, BB)])
    pltpu.sync_copy(dbufL.at[pl.ds(0, BB)], bdst.at[pl.ds(wid * BB, BB)])
    cbuf[0, pl.ds(0, 16)] = jnp.full((16,), cnt, _i32)
    pltpu.sync_copy(cbuf, counts.at[pl.ds(wid, 1)])


_bucket = pl.kernel(
    _bucket_body,
    out_type=(jax.ShapeDtypeStruct((NW * BB,), _i32),
              jax.ShapeDtypeStruct((NW * BB,), _i32),
              jax.ShapeDtypeStruct((NW * BB,), _i32),
              jax.ShapeDtypeStruct((NW, 16), _i32)),
    mesh=_mesh,
    compiler_params=_scparams,
    scratch_types=[
        pltpu.VMEM((BB + 16,), _i32), pltpu.VMEM((BB + 16,), _i32),
        pltpu.VMEM((BB + 16,), _i32),
        pltpu.VMEM((BLK,), _i32), pltpu.VMEM((BLK,), _i32),
        pltpu.VMEM((1, 16), _i32),
    ],
)


def _geom_body(xt, src, dst, rb_out, shm_out, xx, xy, xz, sbuf, dbuf, shblk,
               rbblk):
    base = _wid() * (E // NW)
    pltpu.sync_copy(xt.at[pl.ds(0, 1)], xx)
    pltpu.sync_copy(xt.at[pl.ds(1, 1)], xy)
    pltpu.sync_copy(xt.at[pl.ds(2, 1)], xz)
    z16 = jnp.zeros((16,), _i32)
    iot = lax.iota(_i32, 16)
    centers = iot.astype(_f32) * (RMAX / (NB - 1))
    inv_w = NB / RMAX
    m1x = _mask16(iot, _F1, 1)
    m1y = _mask16(iot, _F1, 2)
    m1z = _mask16(iot, _F1, 3)
    m2x = _mask16(iot, _F2, 1)
    m2y = _mask16(iot, _F2, 2)
    m2z = _mask16(iot, _F2, 3)
    ones = jnp.full((16,), 1.0, _f32)
    zeros = jnp.zeros((16,), _f32)
    base1 = jnp.where(iot == 0, ones, zeros)
    base2 = jnp.where(iot <= 3, ones, zeros)

    def blk(i, carry):
        e0 = base + i * BLK
        pltpu.sync_copy(src.at[pl.ds(e0, BLK)], sbuf)
        pltpu.sync_copy(dst.at[pl.ds(e0, BLK)], dbuf)
        for j in range(BLK // 16):
            si = sbuf[pl.ds(j * 16, 16)]
            di = dbuf[pl.ds(j * 16, 16)]
            vx = (plsc.load_gather(xx, [z16, si])
                  - plsc.load_gather(xx, [z16, di]))
            vy = (plsc.load_gather(xy, [z16, si])
                  - plsc.load_gather(xy, [z16, di]))
            vz = (plsc.load_gather(xz, [z16, si])
                  - plsc.load_gather(xz, [z16, di]))
            d2 = vx * vx + vy * vy + vz * vz + 1e-12
            rs = _rsqrt(d2)
            dist = d2 * rs
            hx = vx * rs
            hy = vy * rs
            hz = vz * rs
            for e2 in range(16):
                e = j * 16 + e2
                xb = jnp.full((16,), hx[e2], _f32)
                yb = jnp.full((16,), hy[e2], _f32)
                zb = jnp.full((16,), hz[e2], _f32)
                f1 = jnp.where(m1x, xb, jnp.where(m1y, yb,
                                                  jnp.where(m1z, zb, base1)))
                f2 = jnp.where(m2x, xb, jnp.where(m2y, yb,
                                                  jnp.where(m2z, zb, base2)))
                shblk[e, :] = f1 * f2
                db = jnp.full((16,), dist[e2], _f32)
                t = (db - centers) * inv_w
                rbblk[e, :] = jnp.exp(-(t * t))
        pltpu.sync_copy(rbblk, rb_out.at[pl.ds(e0, BLK)])
        pltpu.sync_copy(shblk, shm_out.at[pl.ds(e0, BLK)])
        return carry

    lax.fori_loop(0, (E // NW) // BLK, blk, 0)


_geom = pl.kernel(
    _geom_body,
    out_type=(jax.ShapeDtypeStruct((E, NB), _f32),
              jax.ShapeDtypeStruct((E, 16), _f32)),
    mesh=_mesh,
    compiler_params=_scparams,
    scratch_types=[
        pltpu.VMEM((1, N), _f32), pltpu.VMEM((1, N), _f32),
        pltpu.VMEM((1, N), _f32),
        pltpu.VMEM((BLK,), _i32), pltpu.VMEM((BLK,), _i32),
        pltpu.VMEM((BLK, 16), _f32), pltpu.VMEM((BLK, NB), _f32),
    ],
)


def _passa_body(knq, rkp, bsrc, bdst, beid, counts, lout, gout, kqs, kqd,
                rkbuf, sbuf, dbuf, ebuf, lbuf, ctmp, gbuf, sem):
    wid = _wid()
    base = wid * BB
    iot = lax.iota(_i32, 16)
    pltpu.sync_copy(counts, ctmp)
    cv = ctmp[wid, :]
    ninf = jnp.full((16,), -jnp.inf, _f32)

    def blk(i, gv):
        e0 = base + i * BLK
        pltpu.sync_copy(bsrc.at[pl.ds(e0, BLK)], sbuf)
        pltpu.sync_copy(bdst.at[pl.ds(e0, BLK)], dbuf)
        pltpu.sync_copy(beid.at[pl.ds(e0, BLK)], ebuf)
        cp1 = pltpu.async_copy(knq.at[sbuf], kqs, sem)
        cp2 = pltpu.async_copy(knq.at[dbuf], kqd, sem)
        cp3 = pltpu.async_copy(rkp.at[ebuf], rkbuf, sem)
        cp1.wait()
        cp2.wait()
        cp3.wait()
        for j in range(BLK // 16):
            lv = jnp.zeros((16,), _f32)
            for e2 in range(16):
                e = j * 16 + e2
                acc = jnp.zeros((16,), _f32)
                for cc in range(DK // 16):
                    acc = acc + (kqs[e, pl.ds(cc * 16, 16)]
                                 * kqd[e, pl.ds(DK + cc * 16, 16)]
                                 * rkbuf[e, pl.ds(cc * 16, 16)])
                s = jnp.sum(acc) * 0.125
                lv = jnp.where(iot == e2, jnp.full((16,), s, _f32), lv)
            pos = jnp.full((16,), i * BLK + j * 16, _i32) + iot
            lv = jnp.where(pos < cv, lv, ninf)
            lbuf[pl.ds(j * 16, 16)] = lv
            gv = jnp.maximum(gv, lv)
        pltpu.sync_copy(lbuf, lout.at[pl.ds(e0, BLK)])
        return gv

    gv = lax.fori_loop(0, NBB, blk, jnp.full((16,), -jnp.inf, _f32))
    gbuf[0, pl.ds(0, 16)] = jnp.full((16,), jnp.max(gv), _f32)
    pltpu.sync_copy(gbuf, gout.at[pl.ds(wid, 1)])


_passa = pl.kernel(
    _passa_body,
    out_type=(jax.ShapeDtypeStruct((NW * BB,), _f32),
              jax.ShapeDtypeStruct((NW, 16), _f32)),
    mesh=_mesh,
    compiler_params=_scparams,
    scratch_types=[
        pltpu.VMEM((BLK, D), _f32), pltpu.VMEM((BLK, D), _f32),
        pltpu.VMEM((BLK, D), _f32),
        pltpu.VMEM((BLK,), _i32), pltpu.VMEM((BLK,), _i32),
        pltpu.VMEM((BLK,), _i32),
        pltpu.VMEM((BLK,), _f32), pltpu.VMEM((NW, 16), _i32),
        pltpu.VMEM((1, 16), _f32),
        pltpu.SemaphoreType.DMA,
    ],
)


def _passc_body(vn, rv, rkp, lin, gpart, bsrc, bdst, beid, agg, mon, gtmp,
                sbuf, dbuf, ebuf, lbuf, wtmp, dltmp, vbuf, rvbuf, rkpb,
                accT, accM, sem):
    wid = _wid()
    base = wid * BB
    lo = wid * NSEG
    pltpu.sync_copy(gpart, gtmp)
    gv = gtmp[0, :]
    for r in range(1, NW):
        gv = jnp.maximum(gv, gtmp[r, :])
    zv16 = jnp.zeros((16,), _f32)

    def zrow(r, carry):
        for cc in range(D // 16):
            accT[r, pl.ds(cc * 16, 16)] = zv16
        accM[r, :] = zv16
        return carry

    lax.fori_loop(0, ASZ, zrow, 0)

    def blk(i, carry):
        e0 = base + i * BLK
        pltpu.sync_copy(bsrc.at[pl.ds(e0, BLK)], sbuf)
        pltpu.sync_copy(bdst.at[pl.ds(e0, BLK)], dbuf)
        pltpu.sync_copy(beid.at[pl.ds(e0, BLK)], ebuf)
        pltpu.sync_copy(lin.at[pl.ds(e0, BLK)], lbuf)
        cp1 = pltpu.async_copy(vn.at[sbuf], vbuf, sem)
        cp2 = pltpu.async_copy(rv.at[ebuf], rvbuf, sem)
        cp3 = pltpu.async_copy(rkp.at[ebuf], rkpb, sem)
        cp1.wait()
        cp2.wait()
        cp3.wait()
        for j in range(BLK // 16):
            wtmp[pl.ds(j * 16, 16)] = jnp.exp(lbuf[pl.ds(j * 16, 16)] - gv)
            dl = dbuf[pl.ds(j * 16, 16)] - lo
            dl = jnp.minimum(jnp.maximum(dl, 0), ASZ - 1)
            dltmp[pl.ds(j * 16, 16)] = dl

        def edge(e, carry2):
            ef = jnp.full((16,), e, _i32)
            wb = plsc.load_gather(wtmp, [ef])
            r = plsc.load_gather(dltmp, [ef])[0]
            for cc in range(D // 16):
                accT[r, pl.ds(cc * 16, 16)] = (
                    accT[r, pl.ds(cc * 16, 16)]
                    + vbuf[e, pl.ds(cc * 16, 16)]
                    * rvbuf[e, pl.ds(cc * 16, 16)] * wb)
            accM[r, :] = accM[r, :] + rkpb[e, pl.ds(DK, 16)] * wb
            return carry2

        lax.fori_loop(0, BLK, edge, 0)
        return carry

    lax.fori_loop(0, NBB, blk, 0)
    pltpu.sync_copy(accT.at[pl.ds(0, NSEG)], agg.at[pl.ds(lo, NSEG)])
    pltpu.sync_copy(accM.at[pl.ds(0, NSEG)], mon.at[pl.ds(lo, NSEG)])

    @pl.when(wid == NW - 1)
    def _tail():
        pltpu.sync_copy(accT.at[pl.ds(NSEG, 16)],
                        agg.at[pl.ds(NW * NSEG, 16)])
        pltpu.sync_copy(accM.at[pl.ds(NSEG, 16)],
                        mon.at[pl.ds(NW * NSEG, 16)])


_passc = pl.kernel(
    _passc_body,
    out_type=(jax.ShapeDtypeStruct((N, D), _f32),
              jax.ShapeDtypeStruct((N, 16), _f32)),
    mesh=_mesh,
    compiler_params=_scparams,
    scratch_types=[
        pltpu.VMEM((NW, 16), _f32),
        pltpu.VMEM((BLK,), _i32), pltpu.VMEM((BLK,), _i32),
        pltpu.VMEM((BLK,), _i32),
        pltpu.VMEM((BLK,), _f32), pltpu.VMEM((BLK,), _f32),
        pltpu.VMEM((BLK,), _i32),
        pltpu.VMEM((BLK, D), _f32), pltpu.VMEM((BLK, D), _f32),
        pltpu.VMEM((BLK, D), _f32),
        pltpu.VMEM((ASZ, D), _f32), pltpu.VMEM((ASZ, 16), _f32),
        pltpu.SemaphoreType.DMA,
    ],
)


def _mm_body(x_ref, w_ref, o_ref):
    o_ref[...] = jnp.dot(x_ref[...], w_ref[...], precision=_DEF,
                         preferred_element_type=_f32)


def _pallas_mm(x, w, bm=1000):
    m, k = x.shape
    n = w.shape[1]
    return pl.pallas_call(
        _mm_body,
        grid=(m // bm,),
        in_specs=[pl.BlockSpec((bm, k), lambda i: (i, 0)),
                  pl.BlockSpec((k, n), lambda i: (0, 0))],
        out_specs=pl.BlockSpec((bm, n), lambda i: (i, 0)),
        out_shape=jax.ShapeDtypeStruct((m, n), _f32),
    )(x, w)


def _node_body(h_ref, wk_ref, wq_ref, wv_ref, kq_ref, vn_ref):
    h = h_ref[...]
    kq_ref[:, :DK] = jnp.dot(h, wk_ref[...], precision=_DEF,
                             preferred_element_type=_f32)
    kq_ref[:, DK:] = jnp.dot(h, wq_ref[...], precision=_DEF,
                             preferred_element_type=_f32)
    vn_ref[...] = jnp.dot(h, wv_ref[...], precision=_DEF,
                          preferred_element_type=_f32)


def _node(h, wk, wq, wv, bm=2000):
    return pl.pallas_call(
        _node_body,
        grid=(N // bm,),
        in_specs=[pl.BlockSpec((bm, D), lambda i: (i, 0)),
                  pl.BlockSpec((D, DK), lambda i: (0, 0)),
                  pl.BlockSpec((D, DK), lambda i: (0, 0)),
                  pl.BlockSpec((D, D), lambda i: (0, 0))],
        out_specs=[pl.BlockSpec((bm, D), lambda i: (i, 0)),
                   pl.BlockSpec((bm, D), lambda i: (i, 0))],
        out_shape=[jax.ShapeDtypeStruct((N, D), _f32),
                   jax.ShapeDtypeStruct((N, D), _f32)],
    )(h, wk, wq, wv)


def _rkv_body(rb_ref, shm_ref, rkw_ref, rvw_ref, rkp_ref, rv_ref):
    rb = rb_ref[...]
    rkp_ref[:, :DK] = jnp.dot(rb, rkw_ref[...], precision=_DEF,
                              preferred_element_type=_f32)
    rkp_ref[:, DK:DK + 16] = shm_ref[...]
    rkp_ref[:, DK + 16:] = jnp.zeros_like(rkp_ref[:, DK + 16:])
    rv_ref[...] = jnp.dot(rb, rvw_ref[...], precision=_DEF,
                          preferred_element_type=_f32)


def _rkv(rb, shm, rkw, rvw, be=4000):
    return pl.pallas_call(
        _rkv_body,
        grid=(E // be,),
        in_specs=[pl.BlockSpec((be, NB), lambda i: (i, 0)),
                  pl.BlockSpec((be, 16), lambda i: (i, 0)),
                  pl.BlockSpec((NB, DK), lambda i: (0, 0)),
                  pl.BlockSpec((NB, D), lambda i: (0, 0))],
        out_specs=[pl.BlockSpec((be, D), lambda i: (i, 0)),
                   pl.BlockSpec((be, D), lambda i: (i, 0))],
        out_shape=[jax.ShapeDtypeStruct((E, D), _f32),
                   jax.ShapeDtypeStruct((E, D), _f32)],
    )(rb, shm, rkw, rvw)


def _upd_body(agg_ref, mon_ref, h_ref, cm_ref, ssh_ref, wo_ref, o_ref):
    agg = agg_ref[...]
    mon = mon_ref[...]
    den = mon[:, 0:1] + 1e-9
    sh9 = jnp.dot(mon, cm_ref[...], precision=lax.Precision.HIGHEST,
                  preferred_element_type=_f32)
    a128 = (agg + jnp.dot(sh9, ssh_ref[...], precision=_DEF,
                          preferred_element_type=_f32)) / den
    u = a128 * jax.nn.sigmoid(a128)
    o_ref[...] = h_ref[...] + jnp.dot(u, wo_ref[...], precision=_DEF,
                                      preferred_element_type=_f32)


def _upd(agg, mon, h, cm, ssh, wo, bm=2000):
    return pl.pallas_call(
        _upd_body,
        grid=(N // bm,),
        in_specs=[pl.BlockSpec((bm, D), lambda i: (i, 0)),
                  pl.BlockSpec((bm, 16), lambda i: (i, 0)),
                  pl.BlockSpec((bm, D), lambda i: (i, 0)),
                  pl.BlockSpec((16, NSH), lambda i: (0, 0)),
                  pl.BlockSpec((NSH, D), lambda i: (0, 0)),
                  pl.BlockSpec((D, D), lambda i: (0, 0))],
        out_specs=pl.BlockSpec((bm, D), lambda i: (i, 0)),
        out_shape=jax.ShapeDtypeStruct((N, D), _f32),
    )(agg, mon, h, cm, ssh, wo)


def _final_body(h_ref, w1_ref, w2_ref, w3_ref, o_ref):
    h = h_ref[...]
    a = jnp.dot(h, w1_ref[...], precision=_DEF, preferred_element_type=_f32)
    b = jnp.dot(h, w2_ref[...], precision=_DEF, preferred_element_type=_f32)
    o_ref[...] = jnp.dot(a * b, w3_ref[...], precision=_DEF,
                         preferred_element_type=_f32)


def _final(h, w1, w2, w3, bm=2000):
    return pl.pallas_call(
        _final_body,
        grid=(N // bm,),
        in_specs=[pl.BlockSpec((bm, D), lambda i: (i, 0)),
                  pl.BlockSpec((D, D), lambda i: (0, 0)),
                  pl.BlockSpec((D, D), lambda i: (0, 0)),
                  pl.BlockSpec((D, D), lambda i: (0, 0))],
        out_specs=pl.BlockSpec((bm, D), lambda i: (i, 0)),
        out_shape=jax.ShapeDtypeStruct((N, D), _f32),
    )(h, w1, w2, w3)


def kernel(f, x, batch, edge_index, W_in, Wk, Wq, Wv, Rk, Rv, Ssh, Wo, Wr1,
           Wr2, Wr3):
    src = edge_index[0].astype(_i32)
    dst = edge_index[1].astype(_i32)
    xt = x.T
    cm = jnp.asarray(_CM)
    h = _pallas_mm(f, W_in)
    beid, bsrc, bdst, counts = _bucket(src, dst)
    rb, shm = _geom(xt, src, dst)
    for l in range(NL):
        knq, vn = _node(h, Wk[l], Wq[l], Wv[l])
        rkp, rv = _rkv(rb, shm, Rk[l], Rv[l])
        lbkt, gpart = _passa(knq, rkp, bsrc, bdst, beid, counts)
        agg, mon = _passc(vn, rv, rkp, lbkt, gpart, bsrc, bdst, beid)
        h = _upd(agg, mon, h, cm, Ssh[l], Wo[l])
    return _final(h, Wr1, Wr2, Wr3)
